# baseline (speedup 1.0000x reference)
import functools

import jax
import jax.numpy as jnp
from jax.experimental import pallas as pl

_LN_EPS = 1e-5
_ATTN_EPS = 1e-8
_BB = 8
_GG = 8


def _dot(a, b, dims):
    return jax.lax.dot_general(a, b, (dims, ((), ())),
                               preferred_element_type=jnp.float32)


def _body(slots_ref, x_ref, wq_ref, wk_ref, wv_ref, wu_ref, wg_ref,
          out_ref, attn_ref, *, inv_sqrt_d):
    BB, N, D = x_ref.shape
    K = slots_ref.shape[1]
    wq, wk, wv, wu, wg = (wq_ref[...], wk_ref[...], wv_ref[...],
                          wu_ref[...], wg_ref[...])

    ones8_d = jnp.full((8, D), 1.0, dtype=jnp.float32)
    ones8_b = ones8_d.astype(jnp.bfloat16)
    ones_n8 = jnp.full((N, 8), 1.0, dtype=jnp.float32)
    inv_d = 1.0 / D

    ck = _dot(ones8_d, wk, ((1,), (0,)))
    cv = _dot(ones8_d, wv, ((1,), (0,)))[:1]

    s_all = slots_ref[...].reshape(BB * K, D)
    sm = jnp.mean(s_all, axis=-1, keepdims=True)
    sv = jnp.mean((s_all - sm) ** 2, axis=-1, keepdims=True)
    q_all = _dot((s_all - sm) * jax.lax.rsqrt(sv + _LN_EPS), wq, ((1,), (0,)))
    qk_all = _dot(q_all, wk, ((1,), (1,)))
    qck_all = _dot(q_all, ck, ((1,), (1,)))[:, :1]

    def in_stage(grp):
        ssqs, raw24s = {}, {}
        for i in grp:
            xb = x_ref[i].astype(jnp.bfloat16)
            ssqs[i] = _dot(ones8_b, xb * xb, ((1,), (1,)))[:1]
        for i in grp:
            left = jnp.concatenate(
                [qk_all[i * K:(i + 1) * K, :], ones8_d], axis=0)
            raw24s[i] = _dot(left, x_ref[i], ((1,), (1,)))
        return ssqs, raw24s

    groups = [range(g, g + _GG) for g in range(0, BB, _GG)]
    for grp in groups:
        ssqs, raw24s = in_stage(grp)

        attns, ws, zs = {}, {}, {}
        for i in grp:
            mu = raw24s[i][K:K + 1] * inv_d
            iss = jax.lax.rsqrt(ssqs[i] * inv_d - mu * mu + _LN_EPS)
            g1 = iss * inv_sqrt_d
            qck = qck_all[i * K:(i + 1) * K, :]
            e = jnp.exp((g1 * raw24s[i][:K]) - (mu * g1) * qck)
            crinv = 1.0 / jnp.sum(e, axis=0, keepdims=True)
            attn = e * crinv
            attns[i] = attn
            ws[i] = e * (crinv * iss)
            zs[i] = iss * mu
            attn_ref[i] = attn

        rss, ts, ams, adjs = {}, {}, {}, {}
        for i in grp:
            rss[i] = _dot(attns[i], ones_n8, ((1,), (0,)))[:, :1]
            ts[i] = _dot(ws[i], x_ref[i], ((1,), (0,)))
            ams[i] = _dot(attns[i], zs[i], ((1,), (1,)))
            adjs[i] = _dot(attns[i], attns[i], ((1,), (1,)))

        for i in grp:
            rinv = 1.0 / (rss[i] + _ATTN_EPS)
            updates = _dot(ts[i] * rinv, wv, ((1,), (0,))) \
                - (ams[i] * rinv) * cv
            slots_sa = slots_ref[i] + _dot(updates, wu, ((1,), (0,)))
            adj = adjs[i] / (jnp.sum(adjs[i], axis=1, keepdims=True)
                             + _ATTN_EPS)
            agg = _dot(adj, slots_sa, ((1,), (0,)))
            refined = jnp.maximum(_dot(agg, wg, ((1,), (0,))), 0.0)
            out_ref[i] = slots_sa + refined


@jax.jit
def kernel(slots, inputs, Wq, Wk, Wv, Wu, Wg):
    B, K, D = slots.shape
    N = inputs.shape[1] * inputs.shape[2]
    x = inputs.reshape(B, N, D)

    w_spec = pl.BlockSpec((D, D), lambda b: (0, 0))
    out_slots, attn = pl.pallas_call(
        functools.partial(_body, inv_sqrt_d=float(1.0 / (D ** 0.5))),
        grid=(B // _BB,),
        in_specs=[
            pl.BlockSpec((_BB, K, D), lambda b: (b, 0, 0)),
            pl.BlockSpec((_BB, N, D), lambda b: (b, 0, 0)),
            w_spec, w_spec, w_spec, w_spec, w_spec,
        ],
        out_specs=[
            pl.BlockSpec((_BB, K, D), lambda b: (b, 0, 0)),
            pl.BlockSpec((_BB, K, N), lambda b: (b, 0, 0)),
        ],
        out_shape=[
            jax.ShapeDtypeStruct((B, K, D), jnp.float32),
            jax.ShapeDtypeStruct((B, K, N), jnp.float32),
        ],
    )(slots, x, Wq, Wk, Wv, Wu, Wg)
    return out_slots, attn


# speedup vs baseline: 1.0047x; 1.0047x over previous
import functools

import jax
import jax.numpy as jnp
from jax.experimental import pallas as pl

_LN_EPS = 1e-5
_ATTN_EPS = 1e-8
_BB = 32
_GG = 8


def _dot(a, b, dims):
    return jax.lax.dot_general(a, b, (dims, ((), ())),
                               preferred_element_type=jnp.float32)


def _body(slots_ref, x_ref, wq_ref, wk_ref, wv_ref, wu_ref, wg_ref,
          out_ref, attn_ref, *, inv_sqrt_d):
    BB, N, D = x_ref.shape
    K = slots_ref.shape[1]
    wq, wk, wv, wu, wg = (wq_ref[...], wk_ref[...], wv_ref[...],
                          wu_ref[...], wg_ref[...])

    ones8_d = jnp.full((8, D), 1.0, dtype=jnp.float32)
    ones8_b = ones8_d.astype(jnp.bfloat16)
    ones_n8 = jnp.full((N, 8), 1.0, dtype=jnp.float32)
    inv_d = 1.0 / D

    ck = _dot(ones8_d, wk, ((1,), (0,)))
    cv = _dot(ones8_d, wv, ((1,), (0,)))[:1]

    s_all = slots_ref[...].reshape(BB * K, D)
    sm = jnp.mean(s_all, axis=-1, keepdims=True)
    sv = jnp.mean((s_all - sm) ** 2, axis=-1, keepdims=True)
    q_all = _dot((s_all - sm) * jax.lax.rsqrt(sv + _LN_EPS), wq, ((1,), (0,)))
    qk_all = _dot(q_all, wk, ((1,), (1,)))
    qck_all = _dot(q_all, ck, ((1,), (1,)))[:, :1]

    def in_stage(grp):
        ssqs, raw24s = {}, {}
        for i in grp:
            xb = x_ref[i].astype(jnp.bfloat16)
            ssqs[i] = _dot(ones8_b, xb * xb, ((1,), (1,)))[:1]
        for i in grp:
            left = jnp.concatenate(
                [qk_all[i * K:(i + 1) * K, :], ones8_d], axis=0)
            raw24s[i] = _dot(left, x_ref[i], ((1,), (1,)))
        return ssqs, raw24s

    groups = [range(g, g + _GG) for g in range(0, BB, _GG)]
    for grp in groups:
        ssqs, raw24s = in_stage(grp)

        attns, ws, zs = {}, {}, {}
        for i in grp:
            mu = raw24s[i][K:K + 1] * inv_d
            iss = jax.lax.rsqrt(ssqs[i] * inv_d - mu * mu + _LN_EPS)
            g1 = iss * inv_sqrt_d
            qck = qck_all[i * K:(i + 1) * K, :]
            e = jnp.exp((g1 * raw24s[i][:K]) - (mu * g1) * qck)
            crinv = 1.0 / jnp.sum(e, axis=0, keepdims=True)
            attn = e * crinv
            attns[i] = attn
            ws[i] = e * (crinv * iss)
            zs[i] = iss * mu
            attn_ref[i] = attn

        rss, ts, ams, adjs = {}, {}, {}, {}
        for i in grp:
            rss[i] = _dot(attns[i], ones_n8, ((1,), (0,)))[:, :1]
            ts[i] = _dot(ws[i], x_ref[i], ((1,), (0,)))
            ams[i] = _dot(attns[i], zs[i], ((1,), (1,)))
            adjs[i] = _dot(attns[i], attns[i], ((1,), (1,)))

        for i in grp:
            rinv = 1.0 / (rss[i] + _ATTN_EPS)
            updates = _dot(ts[i] * rinv, wv, ((1,), (0,))) \
                - (ams[i] * rinv) * cv
            slots_sa = slots_ref[i] + _dot(updates, wu, ((1,), (0,)))
            adj = adjs[i] / (jnp.sum(adjs[i], axis=1, keepdims=True)
                             + _ATTN_EPS)
            agg = _dot(adj, slots_sa, ((1,), (0,)))
            refined = jnp.maximum(_dot(agg, wg, ((1,), (0,))), 0.0)
            out_ref[i] = slots_sa + refined


@jax.jit
def kernel(slots, inputs, Wq, Wk, Wv, Wu, Wg):
    B, K, D = slots.shape
    N = inputs.shape[1] * inputs.shape[2]
    x = inputs.reshape(B, N, D)

    w_spec = pl.BlockSpec((D, D), lambda b: (0, 0))
    out_slots, attn = pl.pallas_call(
        functools.partial(_body, inv_sqrt_d=float(1.0 / (D ** 0.5))),
        grid=(B // _BB,),
        in_specs=[
            pl.BlockSpec((_BB, K, D), lambda b: (b, 0, 0)),
            pl.BlockSpec((_BB, N, D), lambda b: (b, 0, 0)),
            w_spec, w_spec, w_spec, w_spec, w_spec,
        ],
        out_specs=[
            pl.BlockSpec((_BB, K, D), lambda b: (b, 0, 0)),
            pl.BlockSpec((_BB, K, N), lambda b: (b, 0, 0)),
        ],
        out_shape=[
            jax.ShapeDtypeStruct((B, K, D), jnp.float32),
            jax.ShapeDtypeStruct((B, K, N), jnp.float32),
        ],
    )(slots, x, Wq, Wk, Wv, Wu, Wg)
    return out_slots, attn


# speedup vs baseline: 1.0382x; 1.0334x over previous
import functools

import jax
import jax.numpy as jnp
from jax.experimental import pallas as pl

_LN_EPS = 1e-5
_ATTN_EPS = 1e-8
_BB = 16
_GG = 8


def _dot(a, b, dims):
    return jax.lax.dot_general(a, b, (dims, ((), ())),
                               preferred_element_type=jnp.float32)


def _body(slots_ref, x_ref, wq_ref, wk_ref, wv_ref, wu_ref, wg_ref,
          out_ref, attn_ref, *, inv_sqrt_d):
    BB, N, D = x_ref.shape
    K = slots_ref.shape[1]
    wq, wk, wv, wu, wg = (wq_ref[...], wk_ref[...], wv_ref[...],
                          wu_ref[...], wg_ref[...])

    ones8_d = jnp.full((8, D), 1.0, dtype=jnp.float32)
    ones8_b = ones8_d.astype(jnp.bfloat16)
    ones_n8 = jnp.full((N, 8), 1.0, dtype=jnp.float32)
    inv_d = 1.0 / D

    ck = _dot(ones8_d, wk, ((1,), (0,)))
    cv = _dot(ones8_d, wv, ((1,), (0,)))[:1]

    s_all = slots_ref[...].reshape(BB * K, D)
    sm = jnp.mean(s_all, axis=-1, keepdims=True)
    sv = jnp.mean((s_all - sm) ** 2, axis=-1, keepdims=True)
    q_all = _dot((s_all - sm) * jax.lax.rsqrt(sv + _LN_EPS), wq, ((1,), (0,)))
    qk_all = _dot(q_all, wk, ((1,), (1,)))
    qck_all = _dot(q_all, ck, ((1,), (1,)))[:, :1]

    def in_stage(grp):
        ssqs, raw24s, xbs = {}, {}, {}
        for i in grp:
            xb = x_ref[i].astype(jnp.bfloat16)
            xbs[i] = xb
            ssqs[i] = _dot(ones8_b, xb * xb, ((1,), (1,)))[:1]
        for i in grp:
            left = jnp.concatenate(
                [qk_all[i * K:(i + 1) * K, :], ones8_d], axis=0)
            raw24s[i] = _dot(left, x_ref[i], ((1,), (1,)))
        return ssqs, raw24s, xbs

    groups = [range(g, g + _GG) for g in range(0, BB, _GG)]
    for grp in groups:
        ssqs, raw24s, xbs = in_stage(grp)

        attns, ws, zs = {}, {}, {}
        for i in grp:
            mu = raw24s[i][K:K + 1] * inv_d
            iss = jax.lax.rsqrt(ssqs[i] * inv_d - mu * mu + _LN_EPS)
            g1 = iss * inv_sqrt_d
            qck = qck_all[i * K:(i + 1) * K, :]
            e = jnp.exp((g1 * raw24s[i][:K]) - (mu * g1) * qck)
            crinv = 1.0 / jnp.sum(e, axis=0, keepdims=True)
            attn = e * crinv
            attns[i] = attn
            ws[i] = (e * (crinv * iss)).astype(jnp.bfloat16)
            zs[i] = iss * mu
            attn_ref[i] = attn

        rss, ts, ams, adjs = {}, {}, {}, {}
        for i in grp:
            rss[i] = _dot(attns[i], ones_n8, ((1,), (0,)))[:, :1]
            ts[i] = _dot(ws[i], xbs[i], ((1,), (0,)))
            ams[i] = _dot(attns[i], zs[i], ((1,), (1,)))
            adjs[i] = _dot(attns[i], attns[i], ((1,), (1,)))

        for i in grp:
            rinv = 1.0 / (rss[i] + _ATTN_EPS)
            updates = _dot(ts[i] * rinv, wv, ((1,), (0,))) \
                - (ams[i] * rinv) * cv
            slots_sa = slots_ref[i] + _dot(updates, wu, ((1,), (0,)))
            adj = adjs[i] / (jnp.sum(adjs[i], axis=1, keepdims=True)
                             + _ATTN_EPS)
            agg = _dot(adj, slots_sa, ((1,), (0,)))
            refined = jnp.maximum(_dot(agg, wg, ((1,), (0,))), 0.0)
            out_ref[i] = slots_sa + refined


@jax.jit
def kernel(slots, inputs, Wq, Wk, Wv, Wu, Wg):
    B, K, D = slots.shape
    N = inputs.shape[1] * inputs.shape[2]
    x = inputs.reshape(B, N, D)

    w_spec = pl.BlockSpec((D, D), lambda b: (0, 0))
    out_slots, attn = pl.pallas_call(
        functools.partial(_body, inv_sqrt_d=float(1.0 / (D ** 0.5))),
        grid=(B // _BB,),
        in_specs=[
            pl.BlockSpec((_BB, K, D), lambda b: (b, 0, 0)),
            pl.BlockSpec((_BB, N, D), lambda b: (b, 0, 0)),
            w_spec, w_spec, w_spec, w_spec, w_spec,
        ],
        out_specs=[
            pl.BlockSpec((_BB, K, D), lambda b: (b, 0, 0)),
            pl.BlockSpec((_BB, K, N), lambda b: (b, 0, 0)),
        ],
        out_shape=[
            jax.ShapeDtypeStruct((B, K, D), jnp.float32),
            jax.ShapeDtypeStruct((B, K, N), jnp.float32),
        ],
    )(slots, x, Wq, Wk, Wv, Wu, Wg)
    return out_slots, attn


# speedup vs baseline: 1.5283x; 1.4720x over previous
import functools

import jax
import jax.numpy as jnp
from jax.experimental import pallas as pl

_LN_EPS = 1e-5
_ATTN_EPS = 1e-8
_BB = 16
_GG = 8


def _dot(a, b, dims):
    return jax.lax.dot_general(a, b, (dims, ((), ())),
                               preferred_element_type=jnp.float32)


def _body(slots_ref, x_ref, wq_ref, wk_ref, wv_ref, wu_ref, wg_ref,
          out_ref, attn_ref, *, inv_sqrt_d):
    BB, N, D = x_ref.shape
    K = slots_ref.shape[1]
    wq, wk, wv, wu, wg = (wq_ref[...], wk_ref[...], wv_ref[...],
                          wu_ref[...], wg_ref[...])

    ones8_d = jnp.full((8, D), 1.0, dtype=jnp.float32)
    ones8_b = ones8_d.astype(jnp.bfloat16)
    ones_n8 = jnp.full((N, 8), 1.0, dtype=jnp.float32)
    inv_d = 1.0 / D

    ck = _dot(ones8_d, wk, ((1,), (0,)))
    cv = _dot(ones8_d, wv, ((1,), (0,)))[:1]

    s_all = slots_ref[...].reshape(BB * K, D)
    sm = jnp.mean(s_all, axis=-1, keepdims=True)
    sv = jnp.mean((s_all - sm) ** 2, axis=-1, keepdims=True)
    q_all = _dot((s_all - sm) * jax.lax.rsqrt(sv + _LN_EPS), wq, ((1,), (0,)))
    qk_all = _dot(q_all, wk, ((1,), (1,)))
    qck_all = _dot(q_all, ck, ((1,), (1,)))[:, :1]

    def in_stage(grp):
        ssqs, raw24s, xbs = {}, {}, {}
        for i in grp:
            xb = x_ref[i].astype(jnp.bfloat16)
            xbs[i] = xb
            ssqs[i] = _dot(ones8_b, xb * xb, ((1,), (1,)))[:1]
        for i in grp:
            left = jnp.concatenate(
                [qk_all[i * K:(i + 1) * K, :], ones8_d], axis=0)
            raw24s[i] = _dot(left, x_ref[i], ((1,), (1,)))
        return ssqs, raw24s, xbs

    groups = [range(g, g + _GG) for g in range(0, BB, _GG)]
    for grp in groups:
        ssqs, raw24s, xbs = in_stage(grp)

        attns, ws, zs = {}, {}, {}
        for i in grp:
            mu = raw24s[i][K:K + 1] * inv_d
            iss = jax.lax.rsqrt(ssqs[i] * inv_d - mu * mu + _LN_EPS)
            g1 = iss * inv_sqrt_d
            qck = qck_all[i * K:(i + 1) * K, :]
            e = jnp.exp((g1 * raw24s[i][:K]) - (mu * g1) * qck)
            crinv = 1.0 / jnp.sum(e, axis=0, keepdims=True)
            attn = e * crinv
            attns[i] = attn
            ws[i] = (e * (crinv * iss)).astype(jnp.bfloat16)
            zs[i] = iss * mu
            attn_ref[i] = attn

        GK = _GG * K
        attn_st = jnp.concatenate([attns[i] for i in grp], axis=0)
        m_rows = ([jnp.ones((1, N), jnp.float32)]
                  + [zs[i] for i in grp]
                  + [jnp.zeros((16 - 1 - _GG, N), jnp.float32)])
        m = jnp.concatenate(m_rows, axis=0)
        res = _dot(attn_st, m, ((1,), (1,)))
        rs_st = res[:, :1]
        row_f = jax.lax.broadcasted_iota(jnp.int32, (GK, _GG), 0) // K
        col_f = jax.lax.broadcasted_iota(jnp.int32, (GK, _GG), 1)
        am_st = jnp.sum(jnp.where(row_f == col_f, res[:, 1:_GG + 1], 0.0),
                        axis=1, keepdims=True)

        adj_bd = _dot(attn_st, attn_st, ((1,), (1,)))
        rb = jax.lax.broadcasted_iota(jnp.int32, (GK, GK), 0) // K
        cb = jax.lax.broadcasted_iota(jnp.int32, (GK, GK), 1) // K
        adj_bd = jnp.where(rb == cb, adj_bd, 0.0)
        asum = _dot(adj_bd, jnp.full((GK, 8), 1.0, jnp.float32),
                    ((1,), (0,)))[:, :1]
        adj_bd = adj_bd * (1.0 / (asum + _ATTN_EPS))

        t_st = jnp.concatenate(
            [_dot(ws[i], xbs[i], ((1,), (0,))) for i in grp], axis=0)

        rinv = 1.0 / (rs_st + _ATTN_EPS)
        upd = _dot(t_st * rinv, wv, ((1,), (0,))) - (am_st * rinv) * cv
        s_grp = slots_ref[grp[0]:grp[0] + _GG].reshape(GK, D)
        sa = s_grp + _dot(upd, wu, ((1,), (0,)))
        agg = _dot(adj_bd, sa, ((1,), (0,)))
        refined = jnp.maximum(_dot(agg, wg, ((1,), (0,))), 0.0)
        out_ref[grp[0]:grp[0] + _GG] = (sa + refined).reshape(_GG, K, D)


@jax.jit
def kernel(slots, inputs, Wq, Wk, Wv, Wu, Wg):
    B, K, D = slots.shape
    N = inputs.shape[1] * inputs.shape[2]
    x = inputs.reshape(B, N, D)

    w_spec = pl.BlockSpec((D, D), lambda b: (0, 0))
    out_slots, attn = pl.pallas_call(
        functools.partial(_body, inv_sqrt_d=float(1.0 / (D ** 0.5))),
        grid=(B // _BB,),
        in_specs=[
            pl.BlockSpec((_BB, K, D), lambda b: (b, 0, 0)),
            pl.BlockSpec((_BB, N, D), lambda b: (b, 0, 0)),
            w_spec, w_spec, w_spec, w_spec, w_spec,
        ],
        out_specs=[
            pl.BlockSpec((_BB, K, D), lambda b: (b, 0, 0)),
            pl.BlockSpec((_BB, K, N), lambda b: (b, 0, 0)),
        ],
        out_shape=[
            jax.ShapeDtypeStruct((B, K, D), jnp.float32),
            jax.ShapeDtypeStruct((B, K, N), jnp.float32),
        ],
    )(slots, x, Wq, Wk, Wv, Wu, Wg)
    return out_slots, attn
